# baseline (speedup 1.0000x reference)
import functools

import jax
import jax.numpy as jnp
from jax import lax
from jax.experimental import pallas as pl
from jax.experimental.pallas import tpu as pltpu
from jax.experimental.pallas import tpu_sc as plsc

VOCAB = 32000
HIDDEN = 768
MAX_POS = 64
BATCH = 1024
SEQ = 64

_NC = 2
_NS = 16
_NW = _NC * _NS
_B = BATCH * SEQ
_BPW = _B // _NW
_PPW = SEQ // _NW
_CH = 16
_NCH = _BPW // _CH
_CPP = BATCH // _CH
_NBUF = 8
_VECS = HIDDEN // 16


def _make_sc_kernel():
    mesh = plsc.VectorSubcoreMesh(core_axis_name="c", subcore_axis_name="s")

    @functools.partial(
        pl.kernel,
        mesh=mesh,
        out_type=jax.ShapeDtypeStruct((_B, HIDDEN), jnp.float32),
        scratch_types=[
            pltpu.VMEM((_BPW,), jnp.int32),
            pltpu.VMEM((_PPW, HIDDEN), jnp.float32),
            pltpu.VMEM((_NCH, _CH), jnp.int32),
            pltpu.VMEM((_NBUF, _CH, HIDDEN), jnp.float32),
        ]
        + [pltpu.SemaphoreType.DMA] * (2 * _NBUF),
    )
    def embed(idsT_hbm, tok_hbm, pos_hbm, out_hbm, idx_v, pos_v, oidx_v,
              rows_v, *sems):
        gsem = sems[:_NBUF]
        ssem = sems[_NBUF:]
        wid = lax.axis_index("s") * _NC + lax.axis_index("c")
        base = wid * _BPW
        p0 = wid * _PPW
        pltpu.sync_copy(idsT_hbm.at[pl.ds(base, _BPW)], idx_v)
        pltpu.sync_copy(pos_hbm.at[pl.ds(p0, _PPW)], pos_v)

        lanes = lax.iota(jnp.int32, 16) * SEQ

        def oidx_body(j, carry):
            b0 = lax.rem(j, _CPP) * _CH
            p = p0 + j // _CPP
            for v in range(_CH // 16):
                oidx_v[j, pl.ds(v * 16, 16)] = lanes + ((b0 + v * 16) * SEQ + p)
            return carry

        lax.fori_loop(0, _NCH, oidx_body, 0)

        def gather_start(c, b):
            pltpu.async_copy(
                tok_hbm.at[idx_v.at[pl.ds(c * _CH, _CH)]], rows_v.at[b], gsem[b]
            )

        def gather_wait(b):
            pltpu.make_async_copy(
                tok_hbm.at[pl.ds(0, _CH)], rows_v.at[b], gsem[b]
            ).wait()

        def scatter_start(c, b):
            pltpu.async_copy(
                rows_v.at[b], out_hbm.at[oidx_v.at[c]], ssem[b]
            )

        def scatter_wait(b):
            pltpu.make_async_copy(
                rows_v.at[b], out_hbm.at[pl.ds(0, _CH)], ssem[b]
            ).wait()

        for pb in range(4):
            gather_start(pb, pb)

        def group_body(g, carry):
            for b in range(_NBUF):
                c = g * _NBUF + b
                bp2 = (b + 4) % _NBUF

                @pl.when(c + 4 < _NCH)
                def _():
                    @pl.when(c >= 4)
                    def _():
                        scatter_wait(bp2)

                    gather_start(c + 4, bp2)

                gather_wait(b)
                pl_ = c // _CPP

                for hb in range(2):
                    hoff = hb * (_VECS // 2)
                    pvals = [
                        pos_v[pl_, pl.ds((hoff + h) * 16, 16)]
                        for h in range(_VECS // 2)
                    ]

                    @plsc.parallel_loop(0, _CH, 1, unroll=4)
                    def _(r):
                        for h in range(_VECS // 2):
                            sl = pl.ds((hoff + h) * 16, 16)
                            plsc.addupdate(rows_v.at[b, r, sl], pvals[h])

                scatter_start(c, b)
            return carry

        lax.fori_loop(0, _NCH // _NBUF, group_body, 0)

        for b in range(_NBUF):
            scatter_wait(b)

    return embed


_sc_embed = _make_sc_kernel()


def kernel(input_ids, token_embedding, position_embedding):
    ids_t = input_ids.T.reshape(_B).astype(jnp.int32)
    out = _sc_embed(ids_t, token_embedding, position_embedding)
    return out.reshape(BATCH, SEQ, HIDDEN)


# speedup vs baseline: 1.0359x; 1.0359x over previous
import functools

import jax
import jax.numpy as jnp
from jax import lax
from jax.experimental import pallas as pl
from jax.experimental.pallas import tpu as pltpu
from jax.experimental.pallas import tpu_sc as plsc

VOCAB = 32000
HIDDEN = 768
MAX_POS = 64
BATCH = 1024
SEQ = 64

_NC = 2
_NS = 16
_NW = _NC * _NS
_B = BATCH * SEQ
_BPW = _B // _NW
_PPW = SEQ // _NW
_CH = 32
_NCH = _BPW // _CH
_CPP = BATCH // _CH
_NBUF = 4
_VECS = HIDDEN // 16


def _make_sc_kernel():
    mesh = plsc.VectorSubcoreMesh(core_axis_name="c", subcore_axis_name="s")

    @functools.partial(
        pl.kernel,
        mesh=mesh,
        out_type=jax.ShapeDtypeStruct((_B, HIDDEN), jnp.float32),
        scratch_types=[
            pltpu.VMEM((_BPW,), jnp.int32),
            pltpu.VMEM((_PPW, HIDDEN), jnp.float32),
            pltpu.VMEM((_NCH, _CH), jnp.int32),
            pltpu.VMEM((_NBUF, _CH, HIDDEN), jnp.float32),
        ]
        + [pltpu.SemaphoreType.DMA] * (2 * _NBUF),
    )
    def embed(idsT_hbm, tok_hbm, pos_hbm, out_hbm, idx_v, pos_v, oidx_v,
              rows_v, *sems):
        gsem = sems[:_NBUF]
        ssem = sems[_NBUF:]
        wid = lax.axis_index("s") * _NC + lax.axis_index("c")
        base = wid * _BPW
        p0 = wid * _PPW
        idx_copy = pltpu.async_copy(
            idsT_hbm.at[pl.ds(base, _BPW)], idx_v, gsem[_NBUF - 1]
        )

        lanes = lax.iota(jnp.int32, 16) * SEQ

        def oidx_body(j, carry):
            b0 = lax.rem(j, _CPP) * _CH
            p = p0 + j // _CPP
            for v in range(_CH // 16):
                oidx_v[j, pl.ds(v * 16, 16)] = lanes + ((b0 + v * 16) * SEQ + p)
            return carry

        lax.fori_loop(0, _NCH, oidx_body, 0)
        pltpu.sync_copy(pos_hbm.at[pl.ds(p0, _PPW)], pos_v)
        idx_copy.wait()

        def gather_start(c, b):
            pltpu.async_copy(
                tok_hbm.at[idx_v.at[pl.ds(c * _CH, _CH)]], rows_v.at[b], gsem[b]
            )

        def gather_wait(b):
            pltpu.make_async_copy(
                tok_hbm.at[pl.ds(0, _CH)], rows_v.at[b], gsem[b]
            ).wait()

        def scatter_start(c, b):
            pltpu.async_copy(
                rows_v.at[b], out_hbm.at[oidx_v.at[c]], ssem[b]
            )

        def scatter_wait(b):
            pltpu.make_async_copy(
                rows_v.at[b], out_hbm.at[pl.ds(0, _CH)], ssem[b]
            ).wait()

        gather_start(0, 0)
        gather_start(1, 1)

        def group_body(g, carry):
            for b in range(_NBUF):
                c = g * _NBUF + b
                bp2 = (b + 2) % _NBUF

                @pl.when(c + 2 < _NCH)
                def _():
                    @pl.when(c >= 2)
                    def _():
                        scatter_wait(bp2)

                    gather_start(c + 2, bp2)

                gather_wait(b)
                pl_ = c // _CPP

                for hb in range(2):
                    hoff = hb * (_VECS // 2)
                    pvals = [
                        pos_v[pl_, pl.ds((hoff + h) * 16, 16)]
                        for h in range(_VECS // 2)
                    ]

                    @plsc.parallel_loop(0, _CH, 1, unroll=4)
                    def _(r):
                        for h in range(_VECS // 2):
                            sl = pl.ds((hoff + h) * 16, 16)
                            plsc.addupdate(rows_v.at[b, r, sl], pvals[h])

                scatter_start(c, b)
            return carry

        lax.fori_loop(0, _NCH // _NBUF, group_body, 0)

        for b in range(_NBUF):
            scatter_wait(b)

    return embed


_sc_embed = _make_sc_kernel()


def kernel(input_ids, token_embedding, position_embedding):
    ids_t = input_ids.T.reshape(_B).astype(jnp.int32)
    out = _sc_embed(ids_t, token_embedding, position_embedding)
    return out.reshape(BATCH, SEQ, HIDDEN)
